# Initial kernel scaffold; baseline (speedup 1.0000x reference)
import jax
import jax.numpy as jnp
from jax.experimental import pallas as pl


def kernel(inputs, edge_index, W, b):
    raise NotImplementedError("write your pallas kernel here")


# speedup vs baseline: 2.9965x; 2.9965x over previous
import functools

import jax
import jax.numpy as jnp
from jax import lax
from jax.experimental import pallas as pl
from jax.experimental.pallas import tpu as pltpu
from jax.experimental.pallas import tpu_sc as plsc

N_NODES = 10000
N_EDGES = 320000
D = 128

NC = 2
NS = 16
NW = NC * NS

CH = 128
KJ = 80
EPW = CH * KJ
E_PAD = NW * EPW
TRASH = N_NODES
NP = 10240

_mesh = plsc.VectorSubcoreMesh(core_axis_name="c", subcore_axis_name="s")


def _agg_body(src_hbm, dst_hbm, x_hbm, out_hbm, src_v, dst_v, msg_v, agg_s, sem):
    c = lax.axis_index("c")
    s = lax.axis_index("s")
    wid = c * NS + s

    pltpu.sync_copy(src_hbm.at[wid], src_v)
    pltpu.sync_copy(dst_hbm.at[wid], dst_v)

    zero = jnp.zeros((16,), jnp.float32)

    def zbody(i, carry):
        msg_v[i // 8, pl.ds((i % 8) * 16, 16)] = zero
        return carry

    lax.fori_loop(0, CH * 8, zbody, 0)
    for r in range(NP // NS // CH):
        pltpu.sync_copy(msg_v, agg_s.at[pl.ds(s * (NP // NS) + r * CH, CH)])
    plsc.subcore_barrier()

    def body(j, carry):
        pltpu.async_copy(x_hbm.at[src_v.at[j]], msg_v, sem).wait()
        pltpu.sync_copy(msg_v, agg_s.at[dst_v.at[j]], add=True)
        return carry

    lax.fori_loop(0, KJ, body, 0)
    plsc.subcore_barrier()

    rows = NP // NS
    pltpu.sync_copy(agg_s.at[pl.ds(s * rows, rows)],
                    out_hbm.at[c, pl.ds(s * rows, rows)])


_agg = functools.partial(
    pl.kernel,
    mesh=_mesh,
    out_type=jax.ShapeDtypeStruct((NC, NP, D), jnp.float32),
    scratch_types=[
        pltpu.VMEM((KJ, CH), jnp.int32),
        pltpu.VMEM((KJ, CH), jnp.int32),
        pltpu.VMEM((CH, D), jnp.float32),
        pltpu.VMEM_SHARED((NP, D), jnp.float32),
        pltpu.SemaphoreType.DMA,
    ],
)(_agg_body)


def _mm_body(x_ref, a_ref, wt_ref, b_ref, o_ref):
    h = x_ref[...] + a_ref[0] + a_ref[1]
    y = jnp.dot(h, wt_ref[...], preferred_element_type=jnp.float32)
    o_ref[...] = jnp.maximum(y + b_ref[...], 0.0)


_BN = 1000


def _apply_linear(x, agg, wt, b2):
    grid = N_NODES // _BN
    return pl.pallas_call(
        _mm_body,
        grid=(grid,),
        in_specs=[
            pl.BlockSpec((_BN, D), lambda i: (i, 0)),
            pl.BlockSpec((NC, _BN, D), lambda i: (0, i, 0)),
            pl.BlockSpec((D, D), lambda i: (0, 0)),
            pl.BlockSpec((1, D), lambda i: (0, 0)),
        ],
        out_specs=pl.BlockSpec((_BN, D), lambda i: (i, 0)),
        out_shape=jax.ShapeDtypeStruct((N_NODES, D), jnp.float32),
    )(x, agg, wt, b2)


def kernel(inputs, edge_index, W, b):
    src = edge_index[0].astype(jnp.int32)
    dst = edge_index[1].astype(jnp.int32)
    pad = E_PAD - N_EDGES
    src_p = jnp.concatenate([src, jnp.zeros((pad,), jnp.int32)]).reshape(NW, KJ, CH)
    dst_p = jnp.concatenate([dst, jnp.full((pad,), TRASH, jnp.int32)]).reshape(NW, KJ, CH)
    agg = _agg(src_p, dst_p, inputs)
    return _apply_linear(inputs, agg, W.T, b.reshape(1, D))


# speedup vs baseline: 3.2653x; 1.0897x over previous
import functools

import jax
import jax.numpy as jnp
from jax import lax
from jax.experimental import pallas as pl
from jax.experimental.pallas import tpu as pltpu
from jax.experimental.pallas import tpu_sc as plsc

N_NODES = 10000
N_EDGES = 320000
D = 128

NC = 2
NS = 16
NW = NC * NS

CH = 64
ROUNDS = 4
KJ2 = 40
KJ = ROUNDS * KJ2
EPW = CH * KJ
E_PAD = NW * EPW
TRASH = N_NODES
NP = 10112

_mesh = plsc.VectorSubcoreMesh(core_axis_name="c", subcore_axis_name="s")


NBUF = 4
NGROUP = KJ2 // NBUF


def _agg_body(sd_hbm, x_hbm, out_hbm, idx_v, msg_v, agg_s, gsem, ssem):
    c = lax.axis_index("c")
    s = lax.axis_index("s")
    wid = c * NS + s

    zero = jnp.zeros((16,), jnp.float32)

    def zbody(i, carry):
        msg_v[0, i // 8, pl.ds((i % 8) * 16, 16)] = zero
        return carry

    lax.fori_loop(0, CH * 8, zbody, 0)
    rows = NP // NS
    for r in range(rows // CH):
        pltpu.sync_copy(msg_v.at[0], agg_s.at[pl.ds(s * rows + r * CH, CH)])
    rem = rows % CH
    if rem:
        pltpu.sync_copy(msg_v.at[0].at[pl.ds(0, rem)],
                        agg_s.at[pl.ds(s * rows + (rows // CH) * CH, rem)])
    plsc.subcore_barrier()

    def gather(chunk, b):
        pltpu.async_copy(x_hbm.at[idx_v.at[0, chunk]], msg_v.at[b], gsem.at[b])

    def scatter(chunk, b):
        pltpu.async_copy(msg_v.at[b], agg_s.at[idx_v.at[1, chunk]],
                         ssem.at[b], add=True)

    def gwait(b):
        pltpu.make_async_copy(x_hbm.at[idx_v.at[0, 0]], msg_v.at[b],
                              gsem.at[b]).wait()

    def swait(b):
        pltpu.make_async_copy(msg_v.at[b], agg_s.at[idx_v.at[1, 0]],
                              ssem.at[b]).wait()

    for rnd in range(ROUNDS):
        pltpu.sync_copy(sd_hbm.at[wid, rnd], idx_v)

        for b in range(NBUF):
            gather(b, b)

        def group(g, carry):
            base = g * NBUF
            for b in range(NBUF):
                gwait(b)
                scatter(base + b, b)
            for b in range(NBUF):
                swait(b)
                gather(base + NBUF + b, b)
            return carry

        lax.fori_loop(0, NGROUP - 1, group, 0)

        base = (NGROUP - 1) * NBUF
        for b in range(NBUF):
            gwait(b)
            scatter(base + b, b)
        for b in range(NBUF):
            swait(b)
    plsc.subcore_barrier()

    rows = NP // NS
    pltpu.sync_copy(agg_s.at[pl.ds(s * rows, rows)],
                    out_hbm.at[c, pl.ds(s * rows, rows)])


_agg = functools.partial(
    pl.kernel,
    mesh=_mesh,
    out_type=jax.ShapeDtypeStruct((NC, NP, D), jnp.float32),
    scratch_types=[
        pltpu.VMEM((2, KJ2, CH), jnp.int32),
        pltpu.VMEM((NBUF, CH, D), jnp.float32),
        pltpu.VMEM_SHARED((NP, D), jnp.float32),
        pltpu.SemaphoreType.DMA((NBUF,)),
        pltpu.SemaphoreType.DMA((NBUF,)),
    ],
)(_agg_body)


def _mm_body(x_ref, a_ref, wt_ref, b_ref, o_ref):
    h = x_ref[...] + a_ref[0] + a_ref[1]
    y = jnp.dot(h, wt_ref[...], preferred_element_type=jnp.float32)
    o_ref[...] = jnp.maximum(y + b_ref[...], 0.0)


_BN = 1000


def _apply_linear(x, agg, wt, b2):
    grid = N_NODES // _BN
    return pl.pallas_call(
        _mm_body,
        grid=(grid,),
        in_specs=[
            pl.BlockSpec((_BN, D), lambda i: (i, 0)),
            pl.BlockSpec((NC, _BN, D), lambda i: (0, i, 0)),
            pl.BlockSpec((D, D), lambda i: (0, 0)),
            pl.BlockSpec((1, D), lambda i: (0, 0)),
        ],
        out_specs=pl.BlockSpec((_BN, D), lambda i: (i, 0)),
        out_shape=jax.ShapeDtypeStruct((N_NODES, D), jnp.float32),
    )(x, agg, wt, b2)


def kernel(inputs, edge_index, W, b):
    src = edge_index[0].astype(jnp.int32)
    dst = edge_index[1].astype(jnp.int32)
    pad = E_PAD - N_EDGES
    src_p = jnp.concatenate([src, jnp.zeros((pad,), jnp.int32)])
    dst_p = jnp.concatenate([dst, jnp.full((pad,), TRASH, jnp.int32)])
    sd = jnp.stack([src_p.reshape(NW, ROUNDS, KJ2, CH),
                    dst_p.reshape(NW, ROUNDS, KJ2, CH)], axis=2)
    agg = _agg(sd, inputs)
    return _apply_linear(inputs, agg, W.T, b.reshape(1, D))


# speedup vs baseline: 3.2657x; 1.0001x over previous
import functools

import jax
import jax.numpy as jnp
from jax import lax
from jax.experimental import pallas as pl
from jax.experimental.pallas import tpu as pltpu
from jax.experimental.pallas import tpu_sc as plsc

N_NODES = 10000
N_EDGES = 320000
D = 128

NC = 2
NS = 16
NW = NC * NS

CH = 64
ROUNDS = 4
KJ2 = 40
KJ = ROUNDS * KJ2
EPW = CH * KJ
E_PAD = NW * EPW
TRASH = N_NODES
NP = 10112

_mesh = plsc.VectorSubcoreMesh(core_axis_name="c", subcore_axis_name="s")


NBUF = 4
NGROUP = KJ2 // NBUF


def _agg_body(sd_hbm, x_hbm, out_hbm, idx_v, msg_v, agg_s, gsem, ssem):
    c = lax.axis_index("c")
    s = lax.axis_index("s")
    wid = c * NS + s

    zero = jnp.zeros((16,), jnp.float32)

    def zbody(i, carry):
        msg_v[0, i // 8, pl.ds((i % 8) * 16, 16)] = zero
        return carry

    lax.fori_loop(0, CH * 8, zbody, 0)
    rows = NP // NS
    for r in range(rows // CH):
        pltpu.sync_copy(msg_v.at[0], agg_s.at[pl.ds(s * rows + r * CH, CH)])
    rem = rows % CH
    if rem:
        pltpu.sync_copy(msg_v.at[0].at[pl.ds(0, rem)],
                        agg_s.at[pl.ds(s * rows + (rows // CH) * CH, rem)])
    plsc.subcore_barrier()

    def gather(chunk, b):
        pltpu.async_copy(x_hbm.at[idx_v.at[0, chunk]], msg_v.at[b], gsem.at[b])

    def scatter(chunk, b):
        pltpu.async_copy(msg_v.at[b], agg_s.at[idx_v.at[1, chunk]],
                         ssem.at[b], add=True)

    def gwait(b):
        pltpu.make_async_copy(x_hbm.at[idx_v.at[0, 0]], msg_v.at[b],
                              gsem.at[b]).wait()

    def swait(b):
        pltpu.make_async_copy(msg_v.at[b], agg_s.at[idx_v.at[1, 0]],
                              ssem.at[b]).wait()

    for rnd in range(ROUNDS):
        pltpu.sync_copy(sd_hbm.at[wid, rnd], idx_v)

        for b in range(NBUF):
            gather(b, b)

        def group(g, carry):
            base = g * NBUF
            for b in range(NBUF):
                gwait(b)
                scatter(base + b, b)
            for b in range(NBUF):
                swait(b)
                gather(base + NBUF + b, b)
            return carry

        lax.fori_loop(0, NGROUP - 1, group, 0)

        base = (NGROUP - 1) * NBUF
        for b in range(NBUF):
            gwait(b)
            scatter(base + b, b)
        for b in range(NBUF):
            swait(b)
    plsc.subcore_barrier()

    rows = NP // NS
    pltpu.sync_copy(agg_s.at[pl.ds(s * rows, rows)],
                    out_hbm.at[c, pl.ds(s * rows, rows)])


_agg = functools.partial(
    pl.kernel,
    mesh=_mesh,
    out_type=jax.ShapeDtypeStruct((NC, NP, D), jnp.float32),
    scratch_types=[
        pltpu.VMEM((2, KJ2, CH), jnp.int32),
        pltpu.VMEM((NBUF, CH, D), jnp.float32),
        pltpu.VMEM_SHARED((NP, D), jnp.float32),
        pltpu.SemaphoreType.DMA((NBUF,)),
        pltpu.SemaphoreType.DMA((NBUF,)),
    ],
)(_agg_body)


def _mm_body(x_ref, a_ref, wt_ref, b_ref, o_ref):
    h = x_ref[...] + a_ref[0] + a_ref[1]
    y = jnp.dot(h, wt_ref[...], preferred_element_type=jnp.float32)
    o_ref[...] = jnp.maximum(y + b_ref[...], 0.0)


_BN = 1000


def _apply_linear(x, agg, wt, b2):
    grid = N_NODES // _BN
    return pl.pallas_call(
        _mm_body,
        grid=(grid,),
        in_specs=[
            pl.BlockSpec((_BN, D), lambda i: (i, 0)),
            pl.BlockSpec((NC, _BN, D), lambda i: (0, i, 0)),
            pl.BlockSpec((D, D), lambda i: (0, 0)),
            pl.BlockSpec((1, D), lambda i: (0, 0)),
        ],
        out_specs=pl.BlockSpec((_BN, D), lambda i: (i, 0)),
        out_shape=jax.ShapeDtypeStruct((N_NODES, D), jnp.float32),
    )(x, agg, wt, b2)


def kernel(inputs, edge_index, W, b):
    src = edge_index[0].astype(jnp.int32)
    dst = edge_index[1].astype(jnp.int32)
    pad = E_PAD - N_EDGES
    src_p = jnp.concatenate([src, jnp.zeros((pad,), jnp.int32)])
    trash = TRASH + jnp.arange(pad, dtype=jnp.int32) % (NP - N_NODES)
    dst_p = jnp.concatenate([dst, trash])
    sd = jnp.stack([src_p.reshape(NW, ROUNDS, KJ2, CH),
                    dst_p.reshape(NW, ROUNDS, KJ2, CH)], axis=2)
    agg = _agg(sd, inputs)
    return _apply_linear(inputs, agg, W.T, b.reshape(1, D))


# speedup vs baseline: 3.2685x; 1.0009x over previous
import functools

import jax
import jax.numpy as jnp
from jax import lax
from jax.experimental import pallas as pl
from jax.experimental.pallas import tpu as pltpu
from jax.experimental.pallas import tpu_sc as plsc

N_NODES = 10000
N_EDGES = 320000
D = 128

NC = 2
NS = 16
NW = NC * NS

CH = 64
ROUNDS = 4
KJ2 = 40
KJ = ROUNDS * KJ2
EPW = CH * KJ
E_PAD = NW * EPW
TRASH = N_NODES
NP = 10112

_mesh = plsc.VectorSubcoreMesh(core_axis_name="c", subcore_axis_name="s")


NBUF = 4
NGROUP = KJ2 // NBUF


def _agg_body(sd_hbm, x_hbm, out_hbm, idx_v, msg_v, agg_s, gsem, ssem):
    c = lax.axis_index("c")
    s = lax.axis_index("s")
    wid = c * NS + s

    zero = jnp.zeros((16,), jnp.float32)

    with jax.named_scope("zeroinit"):
        def zbody(i, carry):
            msg_v[0, i // 8, pl.ds((i % 8) * 16, 16)] = zero
            return carry

        lax.fori_loop(0, CH * 8, zbody, 0)
        rows = NP // NS
        for r in range(rows // CH):
            pltpu.sync_copy(msg_v.at[0], agg_s.at[pl.ds(s * rows + r * CH, CH)])
        rem = rows % CH
        if rem:
            pltpu.sync_copy(msg_v.at[0].at[pl.ds(0, rem)],
                            agg_s.at[pl.ds(s * rows + (rows // CH) * CH, rem)])
        plsc.subcore_barrier()

    def gather(chunk, b):
        pltpu.async_copy(x_hbm.at[idx_v.at[0, chunk]], msg_v.at[b], gsem.at[b])

    def scatter(chunk, b):
        pltpu.async_copy(msg_v.at[b], agg_s.at[idx_v.at[1, chunk]],
                         ssem.at[b], add=True)

    def gwait(b):
        pltpu.make_async_copy(x_hbm.at[idx_v.at[0, 0]], msg_v.at[b],
                              gsem.at[b]).wait()

    def swait(b):
        pltpu.make_async_copy(msg_v.at[b], agg_s.at[idx_v.at[1, 0]],
                              ssem.at[b]).wait()

    for rnd in range(ROUNDS):
      with jax.named_scope(f"round{rnd}"):
        pltpu.sync_copy(sd_hbm.at[wid, rnd], idx_v)

        for b in range(NBUF):
            gather(b, b)

        def group(g, carry):
            base = g * NBUF
            for b in range(NBUF):
                gwait(b)
                scatter(base + b, b)
            for b in range(NBUF):
                swait(b)
                gather(base + NBUF + b, b)
            return carry

        lax.fori_loop(0, NGROUP - 1, group, 0)

        base = (NGROUP - 1) * NBUF
        for b in range(NBUF):
            gwait(b)
            scatter(base + b, b)
        for b in range(NBUF):
            swait(b)
    plsc.subcore_barrier()

    with jax.named_scope("writeback"):
        rows = NP // NS
        pltpu.sync_copy(agg_s.at[pl.ds(s * rows, rows)],
                        out_hbm.at[c, pl.ds(s * rows, rows)])


_agg = functools.partial(
    pl.kernel,
    mesh=_mesh,
    out_type=jax.ShapeDtypeStruct((NC, NP, D), jnp.float32),
    scratch_types=[
        pltpu.VMEM((2, KJ2, CH), jnp.int32),
        pltpu.VMEM((NBUF, CH, D), jnp.float32),
        pltpu.VMEM_SHARED((NP, D), jnp.float32),
        pltpu.SemaphoreType.DMA((NBUF,)),
        pltpu.SemaphoreType.DMA((NBUF,)),
    ],
)(_agg_body)


def _mm_body(x_ref, a_ref, wt_ref, b_ref, o_ref):
    h = x_ref[...] + a_ref[0] + a_ref[1]
    y = jnp.dot(h, wt_ref[...], preferred_element_type=jnp.float32)
    o_ref[...] = jnp.maximum(y + b_ref[...], 0.0)


_BN = 1000


def _apply_linear(x, agg, wt, b2):
    grid = N_NODES // _BN
    return pl.pallas_call(
        _mm_body,
        grid=(grid,),
        in_specs=[
            pl.BlockSpec((_BN, D), lambda i: (i, 0)),
            pl.BlockSpec((NC, _BN, D), lambda i: (0, i, 0)),
            pl.BlockSpec((D, D), lambda i: (0, 0)),
            pl.BlockSpec((1, D), lambda i: (0, 0)),
        ],
        out_specs=pl.BlockSpec((_BN, D), lambda i: (i, 0)),
        out_shape=jax.ShapeDtypeStruct((N_NODES, D), jnp.float32),
    )(x, agg, wt, b2)


def kernel(inputs, edge_index, W, b):
    src = edge_index[0].astype(jnp.int32)
    dst = edge_index[1].astype(jnp.int32)
    pad = E_PAD - N_EDGES
    src_p = jnp.concatenate([src, jnp.zeros((pad,), jnp.int32)])
    trash = TRASH + jnp.arange(pad, dtype=jnp.int32) % (NP - N_NODES)
    dst_p = jnp.concatenate([dst, trash])
    sd = jnp.stack([src_p.reshape(NW, ROUNDS, KJ2, CH),
                    dst_p.reshape(NW, ROUNDS, KJ2, CH)], axis=2)
    agg = _agg(sd, inputs)
    return _apply_linear(inputs, agg, W.T, b.reshape(1, D))
